# baseline (speedup 1.0000x reference)
import functools

import jax
import jax.numpy as jnp
from jax.experimental import pallas as pl
from jax.experimental.pallas import tpu as pltpu

N = 10000
D_IN = 128
D_H = 128
D_OUT = 64
BLOCK_R = 400
NB = N // BLOCK_R


def _gcn_kernel(x_ref, adj_ref, w1_ref, b1_ref, w2_ref, b2_ref,
                out_ref, g1_s, g2_s):
    p = pl.program_id(0)
    j = pl.program_id(1)

    @pl.when(jnp.logical_and(p == 0, j == 0))
    def _():
        g1_s[...] = (
            jnp.dot(x_ref[...], w1_ref[...], preferred_element_type=jnp.float32)
            + b1_ref[...]
        )

    @pl.when(p == 0)
    def _():
        h1 = jnp.maximum(
            jnp.dot(adj_ref[...], g1_s[...], preferred_element_type=jnp.float32),
            0.0,
        )
        g2_s[pl.ds(j * BLOCK_R, BLOCK_R), :] = (
            jnp.dot(h1, w2_ref[...], preferred_element_type=jnp.float32)
            + b2_ref[...]
        )
        out_ref[...] = jnp.zeros_like(out_ref)

    @pl.when(p == 1)
    def _():
        out_ref[...] = jnp.dot(
            adj_ref[...], g2_s[...], preferred_element_type=jnp.float32
        )


@functools.partial(jax.jit)
def kernel(x, adj_t, W1, b1, W2, b2):
    b1r = b1.reshape(1, D_H)
    b2r = b2.reshape(1, D_OUT)
    out = pl.pallas_call(
        _gcn_kernel,
        grid=(2, NB),
        in_specs=[
            pl.BlockSpec((N, D_IN), lambda p, j: (0, 0)),
            pl.BlockSpec((BLOCK_R, N), lambda p, j: (j, 0)),
            pl.BlockSpec((D_IN, D_H), lambda p, j: (0, 0)),
            pl.BlockSpec((1, D_H), lambda p, j: (0, 0)),
            pl.BlockSpec((D_H, D_OUT), lambda p, j: (0, 0)),
            pl.BlockSpec((1, D_OUT), lambda p, j: (0, 0)),
        ],
        out_specs=pl.BlockSpec((BLOCK_R, D_OUT), lambda p, j: (j, 0)),
        out_shape=jax.ShapeDtypeStruct((N, D_OUT), jnp.float32),
        scratch_shapes=[
            pltpu.VMEM((N, D_H), jnp.float32),
            pltpu.VMEM((N, D_OUT), jnp.float32),
        ],
    )(x, adj_t, W1, b1r, W2, b2r)
    return out
